# baseline (speedup 1.0000x reference)
import jax
import jax.numpy as jnp
from jax import lax
from jax.experimental import pallas as pl
from jax.experimental.pallas import tpu as pltpu
from jax.experimental.pallas import tpu_sc as plsc

L = 16
SC_ROWS = 2048


def _sc_part(x, emb, n_rows):
    S, B, D = x.shape
    n_src = emb.shape[0]
    NC, NS = 2, 16
    NW = NC * NS
    rows_per_w = n_rows // NW
    CH = 4
    n_chunks = rows_per_w // CH
    NBUF = 4
    LOOKAHEAD = 3
    rows_per_seg = S // n_src
    d_steps = D // L

    mesh = plsc.VectorSubcoreMesh(core_axis_name="c", subcore_axis_name="s")

    def body(x_hbm, emb_hbm, out_hbm, emb_v, b0, b1, b2, b3, *sems):
        bufs = (b0, b1, b2, b3)
        in_sems = sems[:NBUF]
        out_sems = sems[NBUF:]
        wid = lax.axis_index("s") * NC + lax.axis_index("c")
        base = wid * rows_per_w
        pltpu.sync_copy(emb_hbm, emb_v)

        def compute(buf, seg):
            def d_body(j, _):
                off = pl.multiple_of(j * L, L)
                e = emb_v[seg, pl.ds(off, L)]
                for r in range(CH):
                    for b in range(B):
                        buf[r, b, pl.ds(off, L)] += e
                return 0

            lax.fori_loop(0, d_steps, d_body, 0)

        def in_copy(i, b):
            return pltpu.async_copy(
                x_hbm.at[pl.ds(base + i * CH, CH)], bufs[b], in_sems[b])

        def out_copy(i, b):
            return pltpu.async_copy(
                bufs[b], out_hbm.at[pl.ds(base + i * CH, CH)], out_sems[b])

        in_flight = {}
        out_flight = {}
        for i in range(min(LOOKAHEAD, n_chunks)):
            in_flight[i] = in_copy(i, i % NBUF)
        for i in range(n_chunks):
            b = i % NBUF
            j = i + LOOKAHEAD
            if j < n_chunks:
                if j >= NBUF:
                    out_flight.pop(j - NBUF).wait()
                in_flight[j] = in_copy(j, j % NBUF)
            in_flight.pop(i).wait()
            compute(bufs[b], (base + i * CH) // rows_per_seg)
            out_flight[i] = out_copy(i, b)
        for c in out_flight.values():
            c.wait()

    return pl.kernel(
        body,
        out_type=jax.ShapeDtypeStruct((S, B, D), jnp.float32),
        mesh=mesh,
        scratch_types=[
            pltpu.VMEM((n_src, D), jnp.float32),
            pltpu.VMEM((CH, B, D), jnp.float32),
            pltpu.VMEM((CH, B, D), jnp.float32),
            pltpu.VMEM((CH, B, D), jnp.float32),
            pltpu.VMEM((CH, B, D), jnp.float32),
            pltpu.SemaphoreType.DMA,
            pltpu.SemaphoreType.DMA,
            pltpu.SemaphoreType.DMA,
            pltpu.SemaphoreType.DMA,
            pltpu.SemaphoreType.DMA,
            pltpu.SemaphoreType.DMA,
            pltpu.SemaphoreType.DMA,
            pltpu.SemaphoreType.DMA,
        ],
    )(x, emb)


def _tc_part(x, emb, sc_out, row0):
    S, B, D = x.shape
    n_rows = S - row0
    rows_per_seg = S // emb.shape[0]
    block_rows = 512
    n_blocks = n_rows // block_rows
    base_blk = row0 // block_rows
    seg_blks = rows_per_seg // block_rows

    def body(emb_ref, x_ref, acc_ref, o_ref):
        o_ref[...] = x_ref[...] + emb_ref[...]

    return pl.pallas_call(
        body,
        grid=(n_blocks,),
        in_specs=[
            pl.BlockSpec((1, 1, D), lambda k: ((base_blk + k) // seg_blks, 0, 0)),
            pl.BlockSpec((block_rows, B, D), lambda k: (base_blk + k, 0, 0)),
            pl.BlockSpec(memory_space=pl.ANY),
        ],
        out_specs=pl.BlockSpec((block_rows, B, D), lambda k: (base_blk + k, 0, 0)),
        out_shape=jax.ShapeDtypeStruct((S, B, D), jnp.float32),
        input_output_aliases={2: 0},
    )(emb[:, None, :], x, sc_out)


def kernel(x, emb):
    sc_out = _sc_part(x, emb, SC_ROWS)
    return _tc_part(x, emb, sc_out, SC_ROWS)


# speedup vs baseline: 1.0272x; 1.0272x over previous
import jax
import jax.numpy as jnp
from jax import lax
from jax.experimental import pallas as pl
from jax.experimental.pallas import tpu as pltpu
from jax.experimental.pallas import tpu_sc as plsc

L = 16
SC_ROWS = 2048


def _sc_part(x, emb, n_rows):
    S, B, D = x.shape
    n_src = emb.shape[0]
    NC, NS = 2, 16
    NW = NC * NS
    rows_per_w = n_rows // NW
    CH = 8
    n_chunks = rows_per_w // CH
    NBUF = 3
    LOOKAHEAD = 2
    rows_per_seg = S // n_src
    d_steps = D // L

    mesh = plsc.VectorSubcoreMesh(core_axis_name="c", subcore_axis_name="s")

    def body(x_hbm, emb_hbm, out_hbm, emb_v, b0, b1, b2, *sems):
        bufs = (b0, b1, b2)
        in_sems = sems[:NBUF]
        out_sems = sems[NBUF:]
        wid = lax.axis_index("s") * NC + lax.axis_index("c")
        base = wid * rows_per_w
        pltpu.sync_copy(emb_hbm, emb_v)

        def compute(buf, seg):
            def d_body(j, _):
                off = pl.multiple_of(j * L, L)
                e = emb_v[seg, pl.ds(off, L)]
                for r in range(CH):
                    for b in range(B):
                        buf[r, b, pl.ds(off, L)] += e
                return 0

            lax.fori_loop(0, d_steps, d_body, 0)

        def in_copy(i, b):
            return pltpu.async_copy(
                x_hbm.at[pl.ds(base + i * CH, CH)], bufs[b], in_sems[b])

        def out_copy(i, b):
            return pltpu.async_copy(
                bufs[b], out_hbm.at[pl.ds(base + i * CH, CH)], out_sems[b])

        in_flight = {}
        out_flight = {}
        for i in range(min(LOOKAHEAD, n_chunks)):
            in_flight[i] = in_copy(i, i % NBUF)
        for i in range(n_chunks):
            b = i % NBUF
            j = i + LOOKAHEAD
            if j < n_chunks:
                if j >= NBUF:
                    out_flight.pop(j - NBUF).wait()
                in_flight[j] = in_copy(j, j % NBUF)
            in_flight.pop(i).wait()
            compute(bufs[b], (base + i * CH) // rows_per_seg)
            out_flight[i] = out_copy(i, b)
        for c in out_flight.values():
            c.wait()

    return pl.kernel(
        body,
        out_type=jax.ShapeDtypeStruct((S, B, D), jnp.float32),
        mesh=mesh,
        scratch_types=[
            pltpu.VMEM((n_src, D), jnp.float32),
            pltpu.VMEM((CH, B, D), jnp.float32),
            pltpu.VMEM((CH, B, D), jnp.float32),
            pltpu.VMEM((CH, B, D), jnp.float32),
            pltpu.SemaphoreType.DMA,
            pltpu.SemaphoreType.DMA,
            pltpu.SemaphoreType.DMA,
            pltpu.SemaphoreType.DMA,
            pltpu.SemaphoreType.DMA,
            pltpu.SemaphoreType.DMA,
        ],
    )(x, emb)


def _tc_part(x, emb, sc_out, row0):
    S, B, D = x.shape
    n_rows = S - row0
    rows_per_seg = S // emb.shape[0]
    block_rows = 512
    n_blocks = n_rows // block_rows
    base_blk = row0 // block_rows
    seg_blks = rows_per_seg // block_rows

    def body(emb_ref, x_ref, acc_ref, o_ref):
        o_ref[...] = x_ref[...] + emb_ref[...]

    return pl.pallas_call(
        body,
        grid=(n_blocks,),
        in_specs=[
            pl.BlockSpec((1, 1, D), lambda k: ((base_blk + k) // seg_blks, 0, 0)),
            pl.BlockSpec((block_rows, B, D), lambda k: (base_blk + k, 0, 0)),
            pl.BlockSpec(memory_space=pl.ANY),
        ],
        out_specs=pl.BlockSpec((block_rows, B, D), lambda k: (base_blk + k, 0, 0)),
        out_shape=jax.ShapeDtypeStruct((S, B, D), jnp.float32),
        input_output_aliases={2: 0},
    )(emb[:, None, :], x, sc_out)


def kernel(x, emb):
    sc_out = _sc_part(x, emb, SC_ROWS)
    return _tc_part(x, emb, sc_out, SC_ROWS)
